# baseline (speedup 1.0000x reference)
import functools

import jax
import jax.numpy as jnp
from jax import lax
from jax.experimental import pallas as pl
from jax.experimental.pallas import tpu as pltpu
from jax.experimental.pallas import tpu_sc as plsc

_D = 64
_NC = 2
_NS = 16
_NW = _NC * _NS
_K = 512
_SUBI = 128
_SUB = _K // _SUBI


@functools.lru_cache(maxsize=None)
def _make_lookup(B: int, V: int):
    assert B % (_NW * _K) == 0
    n_per_w = B // _NW
    n_chunks = n_per_w // _K
    assert n_chunks % 2 == 0
    n_sub = n_per_w // _SUBI
    mesh = plsc.VectorSubcoreMesh(core_axis_name="c", subcore_axis_name="s")

    @functools.partial(
        pl.kernel,
        out_type=jax.ShapeDtypeStruct((B * _D,), jnp.float32),
        mesh=mesh,
        compiler_params=pltpu.CompilerParams(
            needs_layout_passes=False, use_tc_tiling_on_sc=False),
        scratch_types=[
            pltpu.VMEM((n_sub, _SUBI), jnp.int32),
            pltpu.VMEM((2, _K, _D), jnp.int8),
            pltpu.VMEM((2, _K * _D), jnp.float32),
            pltpu.VMEM((16,), jnp.float32),
            pltpu.SemaphoreType.DMA,
            pltpu.SemaphoreType.DMA,
            pltpu.SemaphoreType.DMA,
            pltpu.SemaphoreType.DMA,
        ],
    )
    def lookup(idx_hbm, tbl_hbm, inv_hbm, out_hbm,
               idx_v, raw_v, out_v, inv_v, gsem0, gsem1, osem0, osem1):
        wid = lax.axis_index("s") * _NC + lax.axis_index("c")
        base0 = wid * n_per_w
        gsem = (gsem0, gsem1)
        osem = (osem0, osem1)
        pltpu.sync_copy(inv_hbm, inv_v)
        pltpu.sync_copy(idx_hbm.at[pl.ds(wid * n_sub, n_sub)], idx_v)
        inv = inv_v[...]

        def issue_gather(g, b):
            for s in range(_SUB):
                pltpu.make_async_copy(
                    tbl_hbm.at[idx_v.at[g * _SUB + s]],
                    raw_v.at[b, pl.ds(s * _SUBI, _SUBI)],
                    gsem[b]).start()

        def wait_gather(b):
            pltpu.make_async_copy(
                tbl_hbm.at[pl.ds(0, _K)], raw_v.at[b], gsem[b]).wait()

        def issue_out(g, b):
            pltpu.make_async_copy(
                out_v.at[b],
                out_hbm.at[pl.ds((base0 + g * _K) * _D, _K * _D)],
                osem[b]).start()

        def wait_out(b):
            pltpu.make_async_copy(
                out_v.at[b], out_hbm.at[pl.ds(0, _K * _D)], osem[b]).wait()

        def compute(b):
            def quad_body(r4, c2):
                for u in range(4):
                    r = r4 * 4 + u
                    words = plsc.bitcast(raw_v[b, r], jnp.int32)
                    for k in range(4):
                        if k < 3:
                            byte = (words << (24 - 8 * k)) >> 24
                        else:
                            byte = words >> 24
                        val = byte.astype(jnp.float32) * inv
                        out_v[b, pl.ds(r * _D + 16 * k, 16)] = val
                return c2
            lax.fori_loop(0, _K // 4, quad_body, 0)

        issue_gather(0, 0)

        def pair_body(t, carry):
            for b in range(2):
                g = 2 * t + b

                @pl.when(g + 1 < n_chunks)
                def _():
                    issue_gather(g + 1, 1 - b)

                wait_gather(b)

                @pl.when(g >= 2)
                def _():
                    wait_out(b)

                compute(b)
                issue_out(g, b)
            return carry

        lax.fori_loop(0, n_chunks // 2, pair_body, 0)
        wait_out(0)
        wait_out(1)

    return lookup


@jax.jit
def kernel(x, weight_quant, c):
    batch, hist = x.shape
    idx = x.reshape(-1, _SUBI).astype(jnp.int32)
    inv = jnp.broadcast_to((1.0 / c).astype(jnp.float32), (16,))
    v = weight_quant.shape[0]
    lookup = _make_lookup(batch * hist, v)
    out = lookup(idx, weight_quant, inv)
    o3 = out.reshape(batch * hist, 4, 16).transpose(0, 2, 1)
    return o3.reshape(batch, hist, _D)


# speedup vs baseline: 2.8261x; 2.8261x over previous
import functools

import jax
import jax.numpy as jnp
from jax import lax
from jax.experimental import pallas as pl
from jax.experimental.pallas import tpu as pltpu
from jax.experimental.pallas import tpu_sc as plsc

_D = 64
_NC = 2
_NS = 16
_NW = _NC * _NS
_K = 512
_SUBI = 128
_SUB = _K // _SUBI


@functools.lru_cache(maxsize=None)
def _make_lookup(B: int, V: int):
    assert B % (_NW * _K) == 0
    n_per_w = B // _NW
    n_chunks = n_per_w // _K
    assert n_chunks % 2 == 0
    n_sub = n_per_w // _SUBI
    mesh = plsc.VectorSubcoreMesh(core_axis_name="c", subcore_axis_name="s")

    @functools.partial(
        pl.kernel,
        out_type=jax.ShapeDtypeStruct((B * _D,), jnp.float32),
        mesh=mesh,
        compiler_params=pltpu.CompilerParams(
            needs_layout_passes=False, use_tc_tiling_on_sc=False),
        scratch_types=[
            pltpu.VMEM((n_sub, _SUBI), jnp.int32),
            pltpu.VMEM((2, _K, _D), jnp.int8),
            pltpu.VMEM((2, _K * _D), jnp.float32),
            pltpu.VMEM((16,), jnp.float32),
            pltpu.SemaphoreType.DMA,
            pltpu.SemaphoreType.DMA,
            pltpu.SemaphoreType.DMA,
            pltpu.SemaphoreType.DMA,
        ],
    )
    def lookup(idx_hbm, tbl_hbm, inv_hbm, out_hbm,
               idx_v, raw_v, out_v, inv_v, gsem0, gsem1, osem0, osem1):
        wid = lax.axis_index("s") * _NC + lax.axis_index("c")
        base0 = wid * n_per_w
        gsem = (gsem0, gsem1)
        osem = (osem0, osem1)
        pltpu.sync_copy(inv_hbm, inv_v)
        pltpu.sync_copy(idx_hbm.at[pl.ds(wid * n_sub, n_sub)], idx_v)
        inv = inv_v[...]
        iot4 = lax.iota(jnp.int32, 16) * 4

        def issue_gather(g, b):
            for s in range(_SUB):
                pltpu.make_async_copy(
                    tbl_hbm.at[idx_v.at[g * _SUB + s]],
                    raw_v.at[b, pl.ds(s * _SUBI, _SUBI)],
                    gsem[b]).start()

        def wait_gather(b):
            pltpu.make_async_copy(
                tbl_hbm.at[pl.ds(0, _K)], raw_v.at[b], gsem[b]).wait()

        def issue_out(g, b):
            pltpu.make_async_copy(
                out_v.at[b],
                out_hbm.at[pl.ds((base0 + g * _K) * _D, _K * _D)],
                osem[b]).start()

        def wait_out(b):
            pltpu.make_async_copy(
                out_v.at[b], out_hbm.at[pl.ds(0, _K * _D)], osem[b]).wait()

        def compute(b):
            def quad_body(r4, c2):
                for u in range(4):
                    r = r4 * 4 + u
                    words = plsc.bitcast(raw_v[b, r], jnp.int32)
                    rbase = iot4 + r * _D
                    for k in range(4):
                        if k < 3:
                            byte = (words << (24 - 8 * k)) >> 24
                        else:
                            byte = words >> 24
                        val = byte.astype(jnp.float32) * inv
                        plsc.store_scatter(out_v.at[b], [rbase + k], val)
                return c2
            lax.fori_loop(0, _K // 4, quad_body, 0)

        issue_gather(0, 0)

        def pair_body(t, carry):
            for b in range(2):
                g = 2 * t + b

                @pl.when(g + 1 < n_chunks)
                def _():
                    issue_gather(g + 1, 1 - b)

                wait_gather(b)

                @pl.when(g >= 2)
                def _():
                    wait_out(b)

                compute(b)
                issue_out(g, b)
            return carry

        lax.fori_loop(0, n_chunks // 2, pair_body, 0)
        wait_out(0)
        wait_out(1)

    return lookup


@jax.jit
def kernel(x, weight_quant, c):
    batch, hist = x.shape
    idx = x.reshape(-1, _SUBI).astype(jnp.int32)
    inv = jnp.broadcast_to((1.0 / c).astype(jnp.float32), (16,))
    v = weight_quant.shape[0]
    lookup = _make_lookup(batch * hist, v)
    out = lookup(idx, weight_quant, inv)
    return out.reshape(batch, hist, _D)


# speedup vs baseline: 2.9365x; 1.0390x over previous
import functools

import jax
import jax.numpy as jnp
from jax import lax
from jax.experimental import pallas as pl
from jax.experimental.pallas import tpu as pltpu
from jax.experimental.pallas import tpu_sc as plsc

_D = 64
_NC = 2
_NS = 16
_NW = _NC * _NS
_K = 512
_SUBI = 128
_SUB = _K // _SUBI


@functools.lru_cache(maxsize=None)
def _make_lookup(NB: int, NH: int, V: int):
    B = NB * NH
    bpw = NB // _NW
    assert bpw == _K and NH % 2 == 0
    mesh = plsc.VectorSubcoreMesh(core_axis_name="c", subcore_axis_name="s")

    @functools.partial(
        pl.kernel,
        out_type=jax.ShapeDtypeStruct((B * _D,), jnp.float32),
        mesh=mesh,
        compiler_params=pltpu.CompilerParams(
            needs_layout_passes=False, use_tc_tiling_on_sc=False),
        scratch_types=[
            pltpu.VMEM((NH, _K), jnp.int32),
            pltpu.VMEM((2, _K, _D), jnp.int8),
            pltpu.VMEM((2, _K * _D), jnp.float32),
            pltpu.VMEM((16,), jnp.float32),
            pltpu.SemaphoreType.DMA,
            pltpu.SemaphoreType.DMA,
            pltpu.SemaphoreType.DMA,
            pltpu.SemaphoreType.DMA,
        ],
    )
    def lookup(idx_hbm, tbl_hbm, inv_hbm, out_hbm,
               idx_v, raw_v, out_v, inv_v, gsem0, gsem1, osem0, osem1):
        wid = lax.axis_index("s") * _NC + lax.axis_index("c")
        gsem = (gsem0, gsem1)
        osem = (osem0, osem1)
        pltpu.sync_copy(inv_hbm, inv_v)
        pltpu.sync_copy(idx_hbm.at[:, pl.ds(wid * _K, _K)], idx_v)
        inv = inv_v[...]
        iot4 = lax.iota(jnp.int32, 16) * 4

        def issue_gather(g, b):
            for s in range(_SUB):
                pltpu.make_async_copy(
                    tbl_hbm.at[idx_v.at[g, pl.ds(s * _SUBI, _SUBI)]],
                    raw_v.at[b, pl.ds(s * _SUBI, _SUBI)],
                    gsem[b]).start()

        def wait_gather(b):
            pltpu.make_async_copy(
                tbl_hbm.at[pl.ds(0, _K)], raw_v.at[b], gsem[b]).wait()

        def issue_out(g, b):
            pltpu.make_async_copy(
                out_v.at[b],
                out_hbm.at[pl.ds((g * NB + wid * _K) * _D, _K * _D)],
                osem[b]).start()

        def wait_out(b):
            pltpu.make_async_copy(
                out_v.at[b], out_hbm.at[pl.ds(0, _K * _D)], osem[b]).wait()

        def compute(b):
            def quad_body(r4, c2):
                for u in range(4):
                    r = r4 * 4 + u
                    words = plsc.bitcast(raw_v[b, r], jnp.int32)
                    rbase = iot4 + r * _D
                    for k in range(4):
                        if k < 3:
                            byte = (words << (24 - 8 * k)) >> 24
                        else:
                            byte = words >> 24
                        val = byte.astype(jnp.float32) * inv
                        plsc.store_scatter(out_v.at[b], [rbase + k], val)
                return c2
            lax.fori_loop(0, _K // 4, quad_body, 0)

        issue_gather(0, 0)

        def pair_body(t, carry):
            for b in range(2):
                g = 2 * t + b

                @pl.when(g + 1 < NH)
                def _():
                    issue_gather(g + 1, 1 - b)

                wait_gather(b)

                @pl.when(g >= 2)
                def _():
                    wait_out(b)

                compute(b)
                issue_out(g, b)
            return carry

        lax.fori_loop(0, NH // 2, pair_body, 0)
        wait_out(0)
        wait_out(1)

    return lookup


@jax.jit
def kernel(x, weight_quant, c):
    batch, hist = x.shape
    idx = x.T.astype(jnp.int32)
    inv = jnp.broadcast_to((1.0 / c).astype(jnp.float32), (16,))
    v = weight_quant.shape[0]
    lookup = _make_lookup(batch, hist, v)
    out = lookup(idx, weight_quant, inv)
    return out.reshape(hist, batch, _D).transpose(1, 0, 2)
